# baseline (speedup 1.0000x reference)
import jax
import jax.numpy as jnp
from jax.experimental import pallas as pl
from jax.experimental.pallas import tpu as pltpu
from jax.experimental.pallas import tpu_sc as plsc

K_NN = 16
N_P = 1000
N_PAD = 1024
X_W = 32
D_W = 128
OUT_W = 65
PAD_COORD = 1e18
GATHER_WINDOW = 128
N_CHUNKS = 4


def _knn_body(xt_ref, x32_ref, wp_ref, wd_ref, b_ref,
              ids_ref, pproj_ref, dpad_ref, key_ref):
    i = pl.program_id(0)
    x32 = x32_ref[0]
    xp = x32[:N_P, :]
    cxc = x32[:, 0:1]
    cyc = x32[:, 1:2]
    cxr = xt_ref[0, 0:1, :N_P]
    cyr = xt_ref[0, 1:2, :N_P]
    dx = cxc - cxr
    dy = cyc - cyr
    d = dx * dx + dy * dy

    rowids = jax.lax.broadcasted_iota(jnp.int32, (N_PAD, N_P), 0)
    colids = jax.lax.broadcasted_iota(jnp.int32, (N_PAD, N_P), 1)
    inf = jnp.float32(jnp.inf)

    bits = jax.lax.bitcast_convert_type(d, jnp.int32)
    keyb = jnp.bitwise_or(jnp.bitwise_and(bits, jnp.int32(~1023)), rowids)
    key = jax.lax.bitcast_convert_type(keyb, jnp.float32)
    key = jnp.where(rowids == colids, inf, key)
    key_ref[...] = jnp.where(rowids >= N_P, inf, key)

    pproj = jnp.dot(xp, wp_ref[...], preferred_element_type=jnp.float32)
    pproj_ref[0] = pproj + b_ref[0]
    dmat = jnp.dot(x32, wd_ref[...], preferred_element_type=jnp.float32)
    dpad_ref[0] = jnp.concatenate(
        [dmat, jnp.zeros((N_PAD, D_W - 64), jnp.float32)], axis=1)

    base = i * N_PAD
    rows = []
    for _ in range(K_NN):
        key = key_ref[...]
        m = jnp.min(key, axis=0, keepdims=True)
        key_ref[...] = jnp.where(key == m, inf, key)
        mb = jax.lax.bitcast_convert_type(m, jnp.int32)
        idr = jnp.bitwise_and(mb, jnp.int32(1023)) + base
        rows.append(jnp.pad(idr, ((0, 0), (0, N_PAD - N_P)),
                            constant_values=base))
    ids_ref[0] = jnp.concatenate(rows, axis=1)


def _sc_gather(d_flat, idx_flat):
    n_idx = idx_flat.shape[1]
    mesh = plsc.VectorSubcoreMesh(core_axis_name="core",
                                  subcore_axis_name="subcore")

    @pl.kernel(
        out_type=jax.ShapeDtypeStruct((n_idx, D_W), jnp.float32),
        mesh=mesh,
    )
    def gather_kernel(x_hbm, i_hbm, o_hbm):
        def body(i_vmem, o_vmem):
            pltpu.sync_copy(x_hbm.at[i_vmem.at[0]], o_vmem)

        pltpu.emit_pipeline(
            body,
            grid=(n_idx // GATHER_WINDOW,),
            in_specs=[pl.BlockSpec((1, GATHER_WINDOW),
                                   index_map=lambda i: (0, i))],
            out_specs=[pl.BlockSpec((GATHER_WINDOW, D_W),
                                    index_map=lambda i: (i, 0))],
            core_axis_name=("core", "subcore"),
            dimension_semantics=(pltpu.PARALLEL,),
        )(i_hbm, o_hbm)

    return gather_kernel(d_flat, idx_flat)


def _edge_mean_body(g_ref, pproj_ref, out_ref):
    g3 = g_ref[0].reshape(K_NN, N_PAD, D_W)
    pp = pproj_ref[0]
    neg = jnp.full((N_P, D_W - 64), -jnp.inf, jnp.float32)
    pp128 = jnp.concatenate([pp, neg], axis=1)
    acc = jnp.zeros((N_P, D_W), jnp.float32)
    for k in range(K_NN):
        acc = acc + jnp.maximum(pp128 + g3[k, :N_P, :], 0.0)
    avg = acc[:, :64] * jnp.float32(1.0 / K_NN)
    ones = jnp.ones((N_P, 1), jnp.float32)
    out_ref[0] = jnp.concatenate([avg, ones], axis=1)


@jax.jit
def kernel(inputs, W, b):
    B_all, N, _ = inputs.shape
    cb = B_all // N_CHUNKS
    stage_a = [_knn_stage(inputs[c * cb:(c + 1) * cb], W, b)
               for c in range(N_CHUNKS)]
    gathered = [_sc_gather(dpad.reshape(cb * N_PAD, D_W), ids)
                for (ids, pproj, dpad) in stage_a]
    outs = [_edge_stage(gathered[c], stage_a[c][1], cb)
            for c in range(N_CHUNKS)]
    return jnp.concatenate(outs, axis=0)


def _knn_stage(inputs, W, b):
    B, N, _ = inputs.shape
    x = inputs[..., :18]
    x32 = jnp.pad(x, ((0, 0), (0, N_PAD - N), (0, X_W - 18)))
    coords_t = jnp.swapaxes(inputs[..., :2], 1, 2)
    xt = jnp.pad(coords_t, ((0, 0), (0, 0), (0, N_PAD - N)),
                 constant_values=PAD_COORD)
    wp = jnp.pad(W[:18] - W[18:], ((0, X_W - 18), (0, 0)))
    wd = jnp.pad(W[18:], ((0, X_W - 18), (0, 0)))
    b2 = b.reshape(1, 64)

    ids, pproj, dpad = pl.pallas_call(
        _knn_body,
        grid=(B,),
        in_specs=[
            pl.BlockSpec((1, 2, N_PAD), lambda i: (i, 0, 0)),
            pl.BlockSpec((1, N_PAD, X_W), lambda i: (i, 0, 0)),
            pl.BlockSpec((X_W, 64), lambda i: (0, 0)),
            pl.BlockSpec((X_W, 64), lambda i: (0, 0)),
            pl.BlockSpec((1, 64), lambda i: (0, 0)),
        ],
        out_specs=[
            pl.BlockSpec((1, 1, K_NN * N_PAD), lambda i: (i, 0, 0)),
            pl.BlockSpec((1, N_P, 64), lambda i: (i, 0, 0)),
            pl.BlockSpec((1, N_PAD, D_W), lambda i: (i, 0, 0)),
        ],
        out_shape=[
            jax.ShapeDtypeStruct((B, 1, K_NN * N_PAD), jnp.int32),
            jax.ShapeDtypeStruct((B, N_P, 64), jnp.float32),
            jax.ShapeDtypeStruct((B, N_PAD, D_W), jnp.float32),
        ],
        scratch_shapes=[pltpu.VMEM((N_PAD, N_P), jnp.float32)],
    )(xt, x32, wp, wd, b2)
    return ids.reshape(1, B * K_NN * N_PAD), pproj, dpad


def _edge_stage(g_flat, pproj, B):
    return pl.pallas_call(
        _edge_mean_body,
        grid=(B,),
        in_specs=[
            pl.BlockSpec((1, K_NN * N_PAD, D_W), lambda i: (i, 0, 0)),
            pl.BlockSpec((1, N_P, 64), lambda i: (i, 0, 0)),
        ],
        out_specs=pl.BlockSpec((1, N_P, OUT_W), lambda i: (i, 0, 0)),
        out_shape=jax.ShapeDtypeStruct((B, N_P, OUT_W), jnp.float32),
    )(g_flat.reshape(B, K_NN * N_PAD, D_W), pproj)


# speedup vs baseline: 1.1474x; 1.1474x over previous
import jax
import jax.numpy as jnp
from jax.experimental import pallas as pl
from jax.experimental.pallas import tpu as pltpu
from jax.experimental.pallas import tpu_sc as plsc

K_NN = 16
N_P = 1000
N_PAD = 1024
X_W = 32
D_W = 128
OUT_W = 65
PAD_COORD = 1e18
GATHER_WINDOW = 128


def _knn_body(xt_ref, x32_ref, wp_ref, wd_ref, b_ref,
              ids_ref, pproj_ref, dpad_ref, key_ref):
    i = pl.program_id(0)
    x32 = x32_ref[0]
    xp = x32[:N_P, :]
    cx = xp[:, 0:1]
    cy = xp[:, 1:2]
    cxt = xt_ref[0, 0:1, :]
    cyt = xt_ref[0, 1:2, :]
    dx = cx - cxt
    dy = cy - cyt
    d = dx * dx + dy * dy

    colids = jax.lax.broadcasted_iota(jnp.int32, (N_P, N_PAD), 1)
    rowids = jax.lax.broadcasted_iota(jnp.int32, (N_P, N_PAD), 0)
    inf = jnp.float32(jnp.inf)

    bits = jax.lax.bitcast_convert_type(d, jnp.int32)
    keyb = jnp.bitwise_or(jnp.bitwise_and(bits, jnp.int32(~1023)), colids)
    key = jax.lax.bitcast_convert_type(keyb, jnp.float32)
    key_ref[...] = jnp.where(colids == rowids, inf, key)

    pproj = jnp.dot(xp, wp_ref[...], preferred_element_type=jnp.float32)
    pproj_ref[0] = pproj + b_ref[0]
    dmat = jnp.dot(x32, wd_ref[...], preferred_element_type=jnp.float32)
    dpad_ref[0] = jnp.concatenate(
        [dmat, jnp.zeros((N_PAD, D_W - 64), jnp.float32)], axis=1)

    base = i * N_PAD
    ids = []
    for _ in range(K_NN):
        key = key_ref[...]
        m = jnp.min(key, axis=1, keepdims=True)
        key_ref[...] = jnp.where(key == m, inf, key)
        mb = jax.lax.bitcast_convert_type(m, jnp.int32)
        ids.append(jnp.bitwise_and(mb, jnp.int32(1023)) + base)
    ids_ref[0] = jnp.concatenate(ids, axis=1)


def _sc_gather(d_flat, idx_flat):
    n_idx = idx_flat.shape[1]
    mesh = plsc.VectorSubcoreMesh(core_axis_name="core",
                                  subcore_axis_name="subcore")

    @pl.kernel(
        out_type=jax.ShapeDtypeStruct((n_idx, D_W), jnp.float32),
        mesh=mesh,
    )
    def gather_kernel(x_hbm, i_hbm, o_hbm):
        def body(i_vmem, o_vmem):
            pltpu.sync_copy(x_hbm.at[i_vmem.at[0]], o_vmem)

        pltpu.emit_pipeline(
            body,
            grid=(n_idx // GATHER_WINDOW,),
            in_specs=[pl.BlockSpec((1, GATHER_WINDOW),
                                   index_map=lambda i: (0, i))],
            out_specs=[pl.BlockSpec((GATHER_WINDOW, D_W),
                                    index_map=lambda i: (i, 0))],
            core_axis_name=("core", "subcore"),
            dimension_semantics=(pltpu.PARALLEL,),
        )(i_hbm, o_hbm)

    return gather_kernel(d_flat, idx_flat)


def _edge_mean_body(g_ref, pproj_ref, out_ref):
    g3 = g_ref[0].reshape(N_P, K_NN, D_W)
    pp = pproj_ref[0]
    neg = jnp.full((N_P, D_W - 64), -jnp.inf, jnp.float32)
    pp128 = jnp.concatenate([pp, neg], axis=1)
    s = jnp.sum(jnp.maximum(pp128[:, None, :] + g3, 0.0), axis=1)
    avg = s[:, :64] * jnp.float32(1.0 / K_NN)
    ones = jnp.ones((N_P, 1), jnp.float32)
    out_ref[0] = jnp.concatenate([avg, ones], axis=1)


N_CHUNKS = 4


@jax.jit
def kernel(inputs, W, b):
    B_all, N, _ = inputs.shape
    cb = B_all // N_CHUNKS
    stage_a = [_knn_stage(inputs[c * cb:(c + 1) * cb], W, b)
               for c in range(N_CHUNKS)]
    gathered = [_sc_gather(dpad.reshape(cb * N_PAD, D_W),
                           ids.reshape(1, cb * N_P * K_NN))
                for (ids, pproj, dpad) in stage_a]
    outs = [_edge_stage(gathered[c], stage_a[c][1], cb)
            for c in range(N_CHUNKS)]
    return jnp.concatenate(outs, axis=0)


def _knn_stage(inputs, W, b):
    B, N, _ = inputs.shape
    x = inputs[..., :18]
    x32 = jnp.pad(x, ((0, 0), (0, N_PAD - N), (0, X_W - 18)))
    coords_t = jnp.swapaxes(inputs[..., :2], 1, 2)
    xt = jnp.pad(coords_t, ((0, 0), (0, 0), (0, N_PAD - N)),
                 constant_values=PAD_COORD)
    wp = jnp.pad(W[:18] - W[18:], ((0, X_W - 18), (0, 0)))
    wd = jnp.pad(W[18:], ((0, X_W - 18), (0, 0)))
    b2 = b.reshape(1, 64)

    ids, pproj, dpad = pl.pallas_call(
        _knn_body,
        grid=(B,),
        in_specs=[
            pl.BlockSpec((1, 2, N_PAD), lambda i: (i, 0, 0)),
            pl.BlockSpec((1, N_PAD, X_W), lambda i: (i, 0, 0)),
            pl.BlockSpec((X_W, 64), lambda i: (0, 0)),
            pl.BlockSpec((X_W, 64), lambda i: (0, 0)),
            pl.BlockSpec((1, 64), lambda i: (0, 0)),
        ],
        out_specs=[
            pl.BlockSpec((1, N_P, K_NN), lambda i: (i, 0, 0)),
            pl.BlockSpec((1, N_P, 64), lambda i: (i, 0, 0)),
            pl.BlockSpec((1, N_PAD, D_W), lambda i: (i, 0, 0)),
        ],
        out_shape=[
            jax.ShapeDtypeStruct((B, N_P, K_NN), jnp.int32),
            jax.ShapeDtypeStruct((B, N_P, 64), jnp.float32),
            jax.ShapeDtypeStruct((B, N_PAD, D_W), jnp.float32),
        ],
        scratch_shapes=[pltpu.VMEM((N_P, N_PAD), jnp.float32)],
    )(xt, x32, wp, wd, b2)

    return ids, pproj, dpad


def _edge_stage(g_flat, pproj, B):
    return pl.pallas_call(
        _edge_mean_body,
        grid=(B,),
        in_specs=[
            pl.BlockSpec((1, N_P * K_NN, D_W), lambda i: (i, 0, 0)),
            pl.BlockSpec((1, N_P, 64), lambda i: (i, 0, 0)),
        ],
        out_specs=pl.BlockSpec((1, N_P, OUT_W), lambda i: (i, 0, 0)),
        out_shape=jax.ShapeDtypeStruct((B, N_P, OUT_W), jnp.float32),
    )(g_flat.reshape(B, N_P * K_NN, D_W), pproj)
